# baseline (speedup 1.0000x reference)
import functools

import jax
import jax.numpy as jnp
from jax import lax
from jax.experimental import pallas as pl
from jax.experimental.pallas import tpu as pltpu
from jax.experimental.pallas import tpu_sc as plsc

VOCAB = 100000
D_MODEL = 1024
SCALE = 32.0

_INFO = plsc.get_sparse_core_info()
_NC, _NS, _L = _INFO.num_cores, _INFO.num_subcores, _INFO.num_lanes
_NW = _NC * _NS
_NBUF = 6
_LOOKAHEAD = 3


def _make_kernel(R, C, D, chunk):
    B = R * C
    assert B % _NW == 0
    b_per_w = B // _NW
    assert C % b_per_w == 0
    w_per_row = C // b_per_w
    assert b_per_w % chunk == 0
    n_chunks = b_per_w // chunk
    slices_per_chunk = chunk * (D // _L)
    cols = D // _L
    col_shift = cols.bit_length() - 1
    mesh = plsc.VectorSubcoreMesh(core_axis_name="c", subcore_axis_name="s")

    @functools.partial(
        pl.kernel,
        mesh=mesh,
        out_type=jax.ShapeDtypeStruct((B, D), jnp.float32),
        scratch_types=[
            pltpu.VMEM((b_per_w,), jnp.int32),
            *[pltpu.VMEM((chunk, D), jnp.float32) for _ in range(_NBUF)],
            *[pltpu.SemaphoreType.DMA for _ in range(2 * _NBUF)],
        ],
    )
    def k(table_hbm, x_hbm, out_hbm, idx_v, *bufs_sems):
        bufs = bufs_sems[:_NBUF]
        gsems = bufs_sems[_NBUF : 2 * _NBUF]
        wsems = bufs_sems[2 * _NBUF :]
        wid = lax.axis_index("s") * _NC + lax.axis_index("c")
        base = wid * b_per_w
        xr = wid // w_per_row
        xc = (wid % w_per_row) * b_per_w
        pltpu.sync_copy(x_hbm.at[xr, pl.ds(xc, b_per_w)], idx_v)

        def gather(g):
            b = g % _NBUF
            return pltpu.async_copy(
                table_hbm.at[idx_v.at[pl.ds(g * chunk, chunk)]], bufs[b], gsems[b]
            )

        def scale(buf):
            @plsc.parallel_loop(0, slices_per_chunk, unroll=8)
            def _(i):
                r = i >> col_shift
                c = (i & (cols - 1)) * _L
                buf[r, pl.ds(c, _L)] = buf[r, pl.ds(c, _L)] * SCALE

        gather_desc = [None] * _NBUF
        write_desc = [None] * _NBUF
        for g in range(min(_LOOKAHEAD, n_chunks)):
            gather_desc[g % _NBUF] = gather(g)
        for g in range(n_chunks):
            b = g % _NBUF
            ahead = g + _LOOKAHEAD
            if ahead < n_chunks:
                ab = ahead % _NBUF
                if write_desc[ab] is not None:
                    write_desc[ab].wait()
                gather_desc[ab] = gather(ahead)
            gather_desc[b].wait()
            scale(bufs[b])
            write_desc[b] = pltpu.async_copy(
                bufs[b], out_hbm.at[pl.ds(base + g * chunk, chunk)], wsems[b]
            )
        for b in range(_NBUF):
            if write_desc[b] is not None:
                write_desc[b].wait()

    return k


@jax.jit
def kernel(x, table):
    R, C = x.shape
    out = _make_kernel(R, C, D_MODEL, 16)(table, x.astype(jnp.int32))
    return out.reshape(R, C, D_MODEL)


# speedup vs baseline: 1.0506x; 1.0506x over previous
import functools

import jax
import jax.numpy as jnp
from jax import lax
from jax.experimental import pallas as pl
from jax.experimental.pallas import tpu as pltpu
from jax.experimental.pallas import tpu_sc as plsc

VOCAB = 100000
D_MODEL = 1024
SCALE = 32.0

_INFO = plsc.get_sparse_core_info()
_NC, _NS, _L = _INFO.num_cores, _INFO.num_subcores, _INFO.num_lanes
_NW = _NC * _NS
_NBUF = 4
_LA = 2


def _make_kernel(R, C, D, chunk):
    B = R * C
    assert B % _NW == 0
    b_per_w = B // _NW
    assert C % b_per_w == 0
    w_per_row = C // b_per_w
    assert b_per_w % chunk == 0
    n_chunks = b_per_w // chunk
    assert n_chunks % _NBUF == 0
    n_groups = n_chunks // _NBUF
    slices_per_chunk = chunk * (D // _L)
    cols = D // _L
    col_shift = cols.bit_length() - 1
    mesh = plsc.VectorSubcoreMesh(core_axis_name="c", subcore_axis_name="s")

    @functools.partial(
        pl.kernel,
        mesh=mesh,
        out_type=jax.ShapeDtypeStruct((B, D), jnp.float32),
        scratch_types=[
            pltpu.VMEM((b_per_w,), jnp.int32),
            *[pltpu.VMEM((chunk, D), jnp.float32) for _ in range(_NBUF)],
            *[pltpu.SemaphoreType.DMA for _ in range(2 * _NBUF)],
        ],
    )
    def k(table_hbm, x_hbm, out_hbm, idx_v, *bufs_sems):
        bufs = bufs_sems[:_NBUF]
        gsems = bufs_sems[_NBUF : 2 * _NBUF]
        wsems = bufs_sems[2 * _NBUF :]
        wid = lax.axis_index("s") * _NC + lax.axis_index("c")
        base = wid * b_per_w
        xr = wid // w_per_row
        xc = (wid % w_per_row) * b_per_w

        def gather(c, b):
            return pltpu.async_copy(
                table_hbm.at[idx_v.at[pl.ds(pl.multiple_of(c * chunk, chunk), chunk)]],
                bufs[b],
                gsems[b],
            )

        def wait_write(b):
            pltpu.make_async_copy(
                out_hbm.at[pl.ds(base, chunk)], bufs[b], wsems[b]
            ).wait()

        def scale(buf):
            @plsc.parallel_loop(0, slices_per_chunk, unroll=8)
            def _(i):
                r = i >> col_shift
                c = (i & (cols - 1)) * _L
                buf[r, pl.ds(c, _L)] = buf[r, pl.ds(c, _L)] * SCALE

        pltpu.sync_copy(x_hbm.at[xr, pl.ds(xc, b_per_w)], idx_v)
        for c in range(_LA):
            gather(c, c)

        def group_body(go, _):
            c0 = go * _NBUF
            for j in range(_NBUF):
                c = c0 + j
                fb = (j + _LA) % _NBUF

                @pl.when(c >= _NBUF - _LA)
                def _():
                    wait_write(fb)

                @pl.when(c + _LA < n_chunks)
                def _():
                    gather(c + _LA, fb)

                pltpu.make_async_copy(
                    out_hbm.at[pl.ds(base, chunk)], bufs[j], gsems[j]
                ).wait()
                scale(bufs[j])
                pltpu.async_copy(
                    bufs[j],
                    out_hbm.at[pl.ds(pl.multiple_of(base + c * chunk, chunk), chunk)],
                    wsems[j],
                )
            return ()

        lax.fori_loop(0, n_groups, group_body, ())
        for c in range(n_chunks - _LA, n_chunks):
            wait_write(c % _NBUF)

    return k


@jax.jit
def kernel(x, table):
    R, C = x.shape
    out = _make_kernel(R, C, D_MODEL, 16)(table, x.astype(jnp.int32))
    return out.reshape(R, C, D_MODEL)


# speedup vs baseline: 1.0507x; 1.0001x over previous
import functools

import jax
import jax.numpy as jnp
from jax import lax
from jax.experimental import pallas as pl
from jax.experimental.pallas import tpu as pltpu
from jax.experimental.pallas import tpu_sc as plsc

VOCAB = 100000
D_MODEL = 1024
SCALE = 32.0

_INFO = plsc.get_sparse_core_info()
_NC, _NS, _L = _INFO.num_cores, _INFO.num_subcores, _INFO.num_lanes
_NW = _NC * _NS
_NBUF = 8
_LA = 4


def _make_kernel(R, C, D, chunk):
    B = R * C
    assert B % _NW == 0
    b_per_w = B // _NW
    assert C % b_per_w == 0
    w_per_row = C // b_per_w
    assert b_per_w % chunk == 0
    n_chunks = b_per_w // chunk
    assert n_chunks % _NBUF == 0
    n_groups = n_chunks // _NBUF
    slices_per_chunk = chunk * (D // _L)
    cols = D // _L
    col_shift = cols.bit_length() - 1
    mesh = plsc.VectorSubcoreMesh(core_axis_name="c", subcore_axis_name="s")

    @functools.partial(
        pl.kernel,
        mesh=mesh,
        out_type=jax.ShapeDtypeStruct((B, D), jnp.float32),
        scratch_types=[
            pltpu.VMEM((b_per_w,), jnp.int32),
            *[pltpu.VMEM((chunk, D), jnp.float32) for _ in range(_NBUF)],
            *[pltpu.SemaphoreType.DMA for _ in range(2 * _NBUF)],
        ],
    )
    def k(table_hbm, x_hbm, out_hbm, idx_v, *bufs_sems):
        bufs = bufs_sems[:_NBUF]
        gsems = bufs_sems[_NBUF : 2 * _NBUF]
        wsems = bufs_sems[2 * _NBUF :]
        wid = lax.axis_index("s") * _NC + lax.axis_index("c")
        base = wid * b_per_w
        xr = wid // w_per_row
        xc = (wid % w_per_row) * b_per_w

        def gather(c, b):
            return pltpu.async_copy(
                table_hbm.at[idx_v.at[pl.ds(pl.multiple_of(c * chunk, chunk), chunk)]],
                bufs[b],
                gsems[b],
            )

        def wait_write(b):
            pltpu.make_async_copy(
                out_hbm.at[pl.ds(base, chunk)], bufs[b], wsems[b]
            ).wait()

        def scale(buf):
            @plsc.parallel_loop(0, slices_per_chunk, unroll=8)
            def _(i):
                r = i >> col_shift
                c = (i & (cols - 1)) * _L
                buf[r, pl.ds(c, _L)] = buf[r, pl.ds(c, _L)] * SCALE

        pltpu.sync_copy(x_hbm.at[xr, pl.ds(xc, b_per_w)], idx_v)
        for c in range(_LA):
            gather(c, c)

        def group_body(go, _):
            c0 = go * _NBUF
            for j in range(_NBUF):
                c = c0 + j
                fb = (j + _LA) % _NBUF

                @pl.when(c >= _NBUF - _LA)
                def _():
                    wait_write(fb)

                @pl.when(c + _LA < n_chunks)
                def _():
                    gather(c + _LA, fb)

                pltpu.make_async_copy(
                    out_hbm.at[pl.ds(base, chunk)], bufs[j], gsems[j]
                ).wait()
                scale(bufs[j])
                pltpu.async_copy(
                    bufs[j],
                    out_hbm.at[pl.ds(pl.multiple_of(base + c * chunk, chunk), chunk)],
                    wsems[j],
                )
            return ()

        lax.fori_loop(0, n_groups, group_body, ())
        for c in range(n_chunks - _LA, n_chunks):
            wait_write(c % _NBUF)

    return k


@jax.jit
def kernel(x, table):
    R, C = x.shape
    out = _make_kernel(R, C, D_MODEL, 8)(table, x.astype(jnp.int32))
    return out.reshape(R, C, D_MODEL)


# speedup vs baseline: 1.0511x; 1.0003x over previous
import functools

import jax
import jax.numpy as jnp
from jax import lax
from jax.experimental import pallas as pl
from jax.experimental.pallas import tpu as pltpu
from jax.experimental.pallas import tpu_sc as plsc

VOCAB = 100000
D_MODEL = 1024
SCALE = 32.0

_INFO = plsc.get_sparse_core_info()
_NC, _NS, _L = _INFO.num_cores, _INFO.num_subcores, _INFO.num_lanes
_NW = _NC * _NS
_NBUF = 4
_LA = 2


def _make_kernel(R, C, D, chunk):
    B = R * C
    assert B % _NW == 0
    b_per_w = B // _NW
    assert C % b_per_w == 0
    w_per_row = C // b_per_w
    assert b_per_w % chunk == 0
    n_chunks = b_per_w // chunk
    assert n_chunks % _NBUF == 0
    n_groups = n_chunks // _NBUF
    slices_per_chunk = chunk * (D // _L)
    cols = D // _L
    col_shift = cols.bit_length() - 1
    mesh = plsc.VectorSubcoreMesh(core_axis_name="c", subcore_axis_name="s")

    @functools.partial(
        pl.kernel,
        mesh=mesh,
        out_type=jax.ShapeDtypeStruct((B, D), jnp.float32),
        scratch_types=[
            pltpu.VMEM((b_per_w,), jnp.int32),
            *[pltpu.VMEM((chunk, D), jnp.float32) for _ in range(_NBUF)],
            *[pltpu.SemaphoreType.DMA for _ in range(2 * _NBUF)],
        ],
    )
    def k(table_hbm, x_hbm, out_hbm, idx_v, *bufs_sems):
        bufs = bufs_sems[:_NBUF]
        gsems = bufs_sems[_NBUF : 2 * _NBUF]
        wsems = bufs_sems[2 * _NBUF :]
        wid = lax.axis_index("s") * _NC + lax.axis_index("c")
        base = wid * b_per_w
        xr = wid // w_per_row
        xc = (wid % w_per_row) * b_per_w

        def gather(c, b):
            return pltpu.async_copy(
                table_hbm.at[idx_v.at[pl.ds(pl.multiple_of(c * chunk, chunk), chunk)]],
                bufs[b],
                gsems[b],
            )

        def wait_write(b):
            pltpu.make_async_copy(
                out_hbm.at[pl.ds(base, chunk)], bufs[b], wsems[b]
            ).wait()

        def scale(buf):
            @plsc.parallel_loop(0, slices_per_chunk, unroll=8)
            def _(i):
                r = i >> col_shift
                c = (i & (cols - 1)) * _L
                buf[r, pl.ds(c, _L)] = buf[r, pl.ds(c, _L)] * SCALE

        pltpu.sync_copy(x_hbm.at[xr, pl.ds(xc, b_per_w)], idx_v)
        for c in range(_LA):
            gather(c, c)

        def group_body(go, _):
            c0 = go * _NBUF
            for j in range(_NBUF):
                c = c0 + j
                fb = (j + _LA) % _NBUF

                @pl.when(c >= _NBUF - _LA)
                def _():
                    wait_write(fb)

                @pl.when(c + _LA < n_chunks)
                def _():
                    gather(c + _LA, fb)

                pltpu.make_async_copy(
                    out_hbm.at[pl.ds(base, chunk)], bufs[j], gsems[j]
                ).wait()
                scale(bufs[j])
                pltpu.async_copy(
                    bufs[j],
                    out_hbm.at[pl.ds(pl.multiple_of(base + c * chunk, chunk), chunk)],
                    wsems[j],
                )
            return ()

        lax.fori_loop(0, n_groups, group_body, ())
        for c in range(n_chunks - _LA, n_chunks):
            wait_write(c % _NBUF)

    return k


@jax.jit
def kernel(x, table):
    R, C = x.shape
    out = _make_kernel(R, C, D_MODEL, 16)(table, x.astype(jnp.int32))
    return out.reshape(R, C, D_MODEL)
